# initial kernel scaffold (unmeasured)
import jax
import jax.numpy as jnp
from jax import lax
from jax.experimental import pallas as pl
from jax.experimental.pallas import tpu as pltpu

N_DEV = 32


def kernel(x, w_mat):
    m, k_local = x.shape
    _, n = w_mat.shape
    chunk = m // N_DEV

    def body(x_ref, w_ref, out_ref, partial_ref, rs_buf, ag_buf,
             rs_send, rs_recv, ag_send, ag_recv):
        me = lax.axis_index("i")
        left = lax.rem(me - 1 + N_DEV, N_DEV)
        right = lax.rem(me + 1, N_DEV)

        barrier_sem = pltpu.get_barrier_semaphore()
        for nbr in (left, right):
            pl.semaphore_signal(
                barrier_sem, inc=1,
                device_id=(nbr,), device_id_type=pl.DeviceIdType.MESH,
            )
        pl.semaphore_wait(barrier_sem, 2)

        x_bf = x_ref[...].astype(jnp.bfloat16)
        w_bf = w_ref[...].astype(jnp.bfloat16)
        partial_ref[...] = lax.dot(
            x_bf, w_bf, preferred_element_type=jnp.float32
        )

        def pchunk(c):
            return partial_ref[pl.ds(c * chunk, chunk), :]

        rs_buf[0, :, :] = pchunk(me).astype(jnp.bfloat16)
        for s in range(N_DEV - 1):
            rdma = pltpu.make_async_remote_copy(
                src_ref=rs_buf.at[s],
                dst_ref=rs_buf.at[s + 1],
                send_sem=rs_send.at[s],
                recv_sem=rs_recv.at[s],
                device_id=(right,),
                device_id_type=pl.DeviceIdType.MESH,
            )
            rdma.start()
            rdma.wait()
            c = lax.rem(me - s - 1 + N_DEV, N_DEV)
            if s < N_DEV - 2:
                rs_buf[s + 1, :, :] = (
                    rs_buf[s + 1, :, :].astype(jnp.float32) + pchunk(c)
                ).astype(jnp.bfloat16)
            else:
                final = jnp.maximum(
                    rs_buf[s + 1, :, :].astype(jnp.float32) + pchunk(c), 0.0
                )
                out_ref[pl.ds(c * chunk, chunk), :] = final
                ag_buf[0, :, :] = final.astype(jnp.bfloat16)

        for h in range(N_DEV - 1):
            rdma = pltpu.make_async_remote_copy(
                src_ref=ag_buf.at[h],
                dst_ref=ag_buf.at[h + 1],
                send_sem=ag_send.at[h],
                recv_sem=ag_recv.at[h],
                device_id=(right,),
                device_id_type=pl.DeviceIdType.MESH,
            )
            rdma.start()
            rdma.wait()
            origin = lax.rem(me - h + N_DEV, N_DEV)
            out_ref[pl.ds(origin * chunk, chunk), :] = (
                ag_buf[h + 1, :, :].astype(jnp.float32)
            )

    return pl.pallas_call(
        body,
        out_shape=jax.ShapeDtypeStruct((m, n), jnp.float32),
        in_specs=[
            pl.BlockSpec(memory_space=pltpu.VMEM),
            pl.BlockSpec(memory_space=pltpu.VMEM),
        ],
        out_specs=pl.BlockSpec(memory_space=pltpu.VMEM),
        scratch_shapes=[
            pltpu.VMEM((m, n), jnp.float32),
            pltpu.VMEM((N_DEV, chunk, n), jnp.bfloat16),
            pltpu.VMEM((N_DEV, chunk, n), jnp.bfloat16),
            pltpu.SemaphoreType.DMA((N_DEV - 1,)),
            pltpu.SemaphoreType.DMA((N_DEV - 1,)),
            pltpu.SemaphoreType.DMA((N_DEV - 1,)),
            pltpu.SemaphoreType.DMA((N_DEV - 1,)),
        ],
        compiler_params=pltpu.CompilerParams(collective_id=0),
    )(x, w_mat)


# baseline (device time: 320799 ns/iter reference)
import jax
import jax.numpy as jnp
from jax import lax
from jax.experimental import pallas as pl
from jax.experimental.pallas import tpu as pltpu

N_DEV = 32


def kernel(x, w_mat):
    m, k_local = x.shape
    _, n = w_mat.shape
    chunk = m // N_DEV

    def body(x_ref, w_ref, out_ref, partial_ref, rs_buf, ag_buf,
             rs_send, rs_recv, ag_send, ag_recv):
        me = lax.axis_index("i")
        left = lax.rem(me - 1 + N_DEV, N_DEV)
        right = lax.rem(me + 1, N_DEV)

        barrier_sem = pltpu.get_barrier_semaphore()
        for nbr in (left, right):
            pl.semaphore_signal(
                barrier_sem, inc=1,
                device_id=(nbr,), device_id_type=pl.DeviceIdType.MESH,
            )
        pl.semaphore_wait(barrier_sem, 2)

        x_bf = x_ref[...].astype(jnp.bfloat16)
        w_bf = w_ref[...].astype(jnp.bfloat16)
        partial_ref[...] = lax.dot(
            x_bf, w_bf, preferred_element_type=jnp.float32
        )

        def pchunk(c):
            return partial_ref[pl.ds(c * chunk, chunk), :]

        rs_buf[0, :, :] = pchunk(me).astype(jnp.bfloat16)
        for s in range(N_DEV - 1):
            rdma = pltpu.make_async_remote_copy(
                src_ref=rs_buf.at[s],
                dst_ref=rs_buf.at[s + 1],
                send_sem=rs_send.at[s],
                recv_sem=rs_recv.at[s],
                device_id=(right,),
                device_id_type=pl.DeviceIdType.MESH,
            )
            rdma.start()
            rdma.wait()
            c = lax.rem(me - s - 1 + N_DEV, N_DEV)
            if s < N_DEV - 2:
                rs_buf[s + 1, :, :] = (
                    rs_buf[s + 1, :, :].astype(jnp.float32) + pchunk(c)
                ).astype(jnp.bfloat16)
            else:
                final = jnp.maximum(
                    rs_buf[s + 1, :, :].astype(jnp.float32) + pchunk(c), 0.0
                )
                out_ref[pl.ds(c * chunk, chunk), :] = final
                ag_buf[0, :, :] = final.astype(jnp.bfloat16)

        for h in range(N_DEV - 1):
            rdma = pltpu.make_async_remote_copy(
                src_ref=ag_buf.at[h],
                dst_ref=ag_buf.at[h + 1],
                send_sem=ag_send.at[h],
                recv_sem=ag_recv.at[h],
                device_id=(right,),
                device_id_type=pl.DeviceIdType.MESH,
            )
            rdma.start()
            rdma.wait()
            origin = lax.rem(me - h + N_DEV, N_DEV)
            out_ref[pl.ds(origin * chunk, chunk), :] = (
                ag_buf[h + 1, :, :].astype(jnp.float32)
            )

    return pl.pallas_call(
        body,
        out_shape=jax.ShapeDtypeStruct((m, n), jnp.float32),
        in_specs=[
            pl.BlockSpec(memory_space=pltpu.VMEM),
            pl.BlockSpec(memory_space=pltpu.VMEM),
        ],
        out_specs=pl.BlockSpec(memory_space=pltpu.VMEM),
        scratch_shapes=[
            pltpu.VMEM((m, n), jnp.float32),
            pltpu.VMEM((N_DEV, chunk, n), jnp.bfloat16),
            pltpu.VMEM((N_DEV, chunk, n), jnp.bfloat16),
            pltpu.SemaphoreType.DMA((N_DEV - 1,)),
            pltpu.SemaphoreType.DMA((N_DEV - 1,)),
            pltpu.SemaphoreType.DMA((N_DEV - 1,)),
            pltpu.SemaphoreType.DMA((N_DEV - 1,)),
        ],
        compiler_params=pltpu.CompilerParams(
            collective_id=0,
            vmem_limit_bytes=100 * 1024 * 1024,
        ),
    )(x, w_mat)


# device time: 171487 ns/iter; 1.8707x vs baseline; 1.8707x over previous
import jax
import jax.numpy as jnp
from jax import lax
from jax.experimental import pallas as pl
from jax.experimental.pallas import tpu as pltpu

N_DEV = 32
NP = 8
NZ = 4
P = [0, 3, 4, 7, 6, 5, 2, 1]


def _sel(idx, table):
    out = jnp.int32(0)
    for j, v in enumerate(table):
        out = jnp.where(idx == j, jnp.int32(v), out)
    return out


def kernel(x, w_mat):
    m, k_local = x.shape
    _, n = w_mat.shape
    piece = m // NP
    sub = piece // NZ
    nh = n // 2

    def body(x_ref, w_ref, out_ref, partial_ref,
             a_buf0, a_buf1, c_buf0, c_buf1, own0, own1,
             zs1_send0, zs1_recv0, zs1_send1, zs1_recv1,
             zs2_send0, zs2_recv0, zs2_send1, zs2_recv1,
             zg2_send0, zg2_recv0, zg2_send1, zg2_recv1,
             zg1_send0, zg1_recv0, zg1_send1, zg1_recv1,
             a_s0, a_r0, a_s1, a_r1, z_s, z_r, c_s0, c_r0, c_s1, c_r1):
        me = lax.axis_index("i")
        z = lax.div(me, NP)
        q = lax.rem(me, NP)
        p = _sel(q, [P.index(j) for j in range(NP)])
        right_q = _sel(p, [P[(j + 1) % NP] for j in range(NP)])
        left_q = _sel(p, [P[(j - 1) % NP] for j in range(NP)])
        right = z * NP + right_q
        left = z * NP + left_q
        partner16 = jnp.where(z < 2, me + 2 * NP, me - 2 * NP)
        partner8 = jnp.where(lax.rem(z, 2) == 0, me + NP, me - NP)

        barrier_sem = pltpu.get_barrier_semaphore()
        for nbr in (left, right, partner16, partner8):
            pl.semaphore_signal(
                barrier_sem, inc=1,
                device_id=(nbr,), device_id_type=pl.DeviceIdType.MESH,
            )
        pl.semaphore_wait(barrier_sem, 4)

        x_bf = x_ref[...].astype(jnp.bfloat16)
        w_bf = w_ref[...].astype(jnp.bfloat16)
        partial_ref[...] = lax.dot(
            x_bf, w_bf, preferred_element_type=jnp.float32
        )

        def ppiece(c, lo, hi):
            return partial_ref[pl.ds(c * piece, piece), lo:hi]

        def copy(src_ref, dst_ref, ssem, rsem, target):
            rdma = pltpu.make_async_remote_copy(
                src_ref=src_ref, dst_ref=dst_ref,
                send_sem=ssem, recv_sem=rsem,
                device_id=(target,), device_id_type=pl.DeviceIdType.MESH,
            )
            rdma.start()
            return rdma

        a_buf0[0, :, :] = ppiece(p, 0, nh).astype(jnp.bfloat16)
        a_buf1[0, :, :] = ppiece(p, nh, n).astype(jnp.bfloat16)
        for s in range(NP - 1):
            r0 = copy(a_buf0.at[s], a_buf0.at[s + 1], a_s0.at[s], a_r0.at[s],
                      right)
            r1 = copy(a_buf1.at[s], a_buf1.at[s + 1], a_s1.at[s], a_r1.at[s],
                      left)
            r0.wait()
            r1.wait()
            c0 = lax.rem(p - s - 1 + NP, NP)
            c1 = lax.rem(p + s + 1, NP)
            if s < NP - 2:
                a_buf0[s + 1, :, :] = (
                    a_buf0[s + 1, :, :].astype(jnp.float32) + ppiece(c0, 0, nh)
                ).astype(jnp.bfloat16)
                a_buf1[s + 1, :, :] = (
                    a_buf1[s + 1, :, :].astype(jnp.float32) + ppiece(c1, nh, n)
                ).astype(jnp.bfloat16)
            else:
                own0[...] = (
                    a_buf0[s + 1, :, :].astype(jnp.float32) + ppiece(c0, 0, nh)
                )
                own1[...] = (
                    a_buf1[s + 1, :, :].astype(jnp.float32) + ppiece(c1, nh, n)
                )

        r_own0 = lax.rem(p + 1, NP)
        r_own1 = lax.rem(p - 1 + NP, NP)

        half = piece // 2
        keep_off = jnp.where(z < 2, 0, half)
        send_off = jnp.where(z < 2, half, 0)
        zs1_send0[...] = own0[pl.ds(send_off, half), :].astype(jnp.bfloat16)
        zs1_send1[...] = own1[pl.ds(send_off, half), :].astype(jnp.bfloat16)
        e0 = copy(zs1_send0, zs1_recv0, z_s.at[0], z_r.at[0], partner16)
        e1 = copy(zs1_send1, zs1_recv1, z_s.at[1], z_r.at[1], partner16)
        e0.wait()
        e1.wait()
        own0[pl.ds(keep_off, half), :] = (
            own0[pl.ds(keep_off, half), :] + zs1_recv0[...].astype(jnp.float32)
        )
        own1[pl.ds(keep_off, half), :] = (
            own1[pl.ds(keep_off, half), :] + zs1_recv1[...].astype(jnp.float32)
        )
        keep2_off = z * sub
        bit0 = lax.rem(z, 2)
        send2_off = jnp.where(bit0 == 0, keep2_off + sub, keep2_off - sub)
        zs2_send0[...] = own0[pl.ds(send2_off, sub), :].astype(jnp.bfloat16)
        zs2_send1[...] = own1[pl.ds(send2_off, sub), :].astype(jnp.bfloat16)
        e0 = copy(zs2_send0, zs2_recv0, z_s.at[2], z_r.at[2], partner8)
        e1 = copy(zs2_send1, zs2_recv1, z_s.at[3], z_r.at[3], partner8)
        e0.wait()
        e1.wait()
        fin0 = jnp.maximum(
            own0[pl.ds(keep2_off, sub), :] + zs2_recv0[...].astype(jnp.float32),
            0.0,
        )
        fin1 = jnp.maximum(
            own1[pl.ds(keep2_off, sub), :] + zs2_recv1[...].astype(jnp.float32),
            0.0,
        )
        zg2_send0[...] = fin0.astype(jnp.bfloat16)
        zg2_send1[...] = fin1.astype(jnp.bfloat16)
        e0 = copy(zg2_send0, zg2_recv0, z_s.at[4], z_r.at[4], partner8)
        e1 = copy(zg2_send1, zg2_recv1, z_s.at[5], z_r.at[5], partner8)
        e0.wait()
        e1.wait()
        my_rel = bit0 * sub
        other_rel = (1 - bit0) * sub
        zg1_send0[pl.ds(my_rel, sub), :] = zg2_send0[...]
        zg1_send0[pl.ds(other_rel, sub), :] = zg2_recv0[...]
        zg1_send1[pl.ds(my_rel, sub), :] = zg2_send1[...]
        zg1_send1[pl.ds(other_rel, sub), :] = zg2_recv1[...]
        e0 = copy(zg1_send0, zg1_recv0, z_s.at[6], z_r.at[6], partner16)
        e1 = copy(zg1_send1, zg1_recv1, z_s.at[7], z_r.at[7], partner16)
        e0.wait()
        e1.wait()
        c_buf0[0, pl.ds(keep_off, half), :] = zg1_send0[...]
        c_buf0[0, pl.ds(send_off, half), :] = zg1_recv0[...]
        c_buf1[0, pl.ds(keep_off, half), :] = zg1_send1[...]
        c_buf1[0, pl.ds(send_off, half), :] = zg1_recv1[...]
        out_ref[pl.ds(r_own0 * piece, piece), 0:nh] = (
            c_buf0[0, :, :].astype(jnp.float32)
        )
        out_ref[pl.ds(r_own1 * piece, piece), nh:n] = (
            c_buf1[0, :, :].astype(jnp.float32)
        )

        for h in range(NP - 1):
            r0 = copy(c_buf0.at[h], c_buf0.at[h + 1], c_s0.at[h], c_r0.at[h],
                      right)
            r1 = copy(c_buf1.at[h], c_buf1.at[h + 1], c_s1.at[h], c_r1.at[h],
                      left)
            r0.wait()
            r1.wait()
            g0 = lax.rem(p - h + NP, NP)
            g1 = lax.rem(p + h, NP)
            out_ref[pl.ds(g0 * piece, piece), 0:nh] = (
                c_buf0[h + 1, :, :].astype(jnp.float32)
            )
            out_ref[pl.ds(g1 * piece, piece), nh:n] = (
                c_buf1[h + 1, :, :].astype(jnp.float32)
            )

    return pl.pallas_call(
        body,
        out_shape=jax.ShapeDtypeStruct((m, n), jnp.float32),
        in_specs=[
            pl.BlockSpec(memory_space=pltpu.VMEM),
            pl.BlockSpec(memory_space=pltpu.VMEM),
        ],
        out_specs=pl.BlockSpec(memory_space=pltpu.VMEM),
        scratch_shapes=[
            pltpu.VMEM((m, n), jnp.float32),
            pltpu.VMEM((NP, piece, nh), jnp.bfloat16),
            pltpu.VMEM((NP, piece, nh), jnp.bfloat16),
            pltpu.VMEM((NP, piece, nh), jnp.bfloat16),
            pltpu.VMEM((NP, piece, nh), jnp.bfloat16),
            pltpu.VMEM((piece, nh), jnp.float32),
            pltpu.VMEM((piece, nh), jnp.float32),
            pltpu.VMEM((piece // 2, nh), jnp.bfloat16),
            pltpu.VMEM((piece // 2, nh), jnp.bfloat16),
            pltpu.VMEM((piece // 2, nh), jnp.bfloat16),
            pltpu.VMEM((piece // 2, nh), jnp.bfloat16),
            pltpu.VMEM((sub, nh), jnp.bfloat16),
            pltpu.VMEM((sub, nh), jnp.bfloat16),
            pltpu.VMEM((sub, nh), jnp.bfloat16),
            pltpu.VMEM((sub, nh), jnp.bfloat16),
            pltpu.VMEM((sub, nh), jnp.bfloat16),
            pltpu.VMEM((sub, nh), jnp.bfloat16),
            pltpu.VMEM((sub, nh), jnp.bfloat16),
            pltpu.VMEM((sub, nh), jnp.bfloat16),
            pltpu.VMEM((piece // 2, nh), jnp.bfloat16),
            pltpu.VMEM((piece // 2, nh), jnp.bfloat16),
            pltpu.VMEM((piece // 2, nh), jnp.bfloat16),
            pltpu.VMEM((piece // 2, nh), jnp.bfloat16),
            pltpu.SemaphoreType.DMA((NP - 1,)),
            pltpu.SemaphoreType.DMA((NP - 1,)),
            pltpu.SemaphoreType.DMA((NP - 1,)),
            pltpu.SemaphoreType.DMA((NP - 1,)),
            pltpu.SemaphoreType.DMA((8,)),
            pltpu.SemaphoreType.DMA((8,)),
            pltpu.SemaphoreType.DMA((NP - 1,)),
            pltpu.SemaphoreType.DMA((NP - 1,)),
            pltpu.SemaphoreType.DMA((NP - 1,)),
            pltpu.SemaphoreType.DMA((NP - 1,)),
        ],
        compiler_params=pltpu.CompilerParams(
            collective_id=0,
            vmem_limit_bytes=100 * 1024 * 1024,
        ),
    )(x, w_mat)


# device time: 145110 ns/iter; 2.2107x vs baseline; 1.1818x over previous
import jax
import jax.numpy as jnp
from jax import lax
from jax.experimental import pallas as pl
from jax.experimental.pallas import tpu as pltpu

N_DEV = 32
NP = 8
NZ = 4
NSUB = 2
P = [0, 3, 4, 7, 6, 5, 2, 1]


def _sel(idx, table):
    out = jnp.int32(0)
    for j, v in enumerate(table):
        out = jnp.where(idx == j, jnp.int32(v), out)
    return out


def kernel(x, w_mat):
    m, k_local = x.shape
    _, n = w_mat.shape
    piece = m // NP
    subrows = piece // NSUB
    sub = piece // NZ
    nh = n // 2

    def body(x_ref, w_ref, out_ref, partial_ref,
             a_buf0, a_buf1, c_buf0, c_buf1, own0, own1,
             zs1_send0, zs1_recv0, zs1_send1, zs1_recv1,
             zs2_send0, zs2_recv0, zs2_send1, zs2_recv1,
             zg2_send0, zg2_recv0, zg2_send1, zg2_recv1,
             zg1_send0, zg1_recv0, zg1_send1, zg1_recv1,
             a_s0, a_r0, a_s1, a_r1, z_s, z_r, c_s0, c_r0, c_s1, c_r1):
        me = lax.axis_index("i")
        z = lax.div(me, NP)
        q = lax.rem(me, NP)
        p = _sel(q, [P.index(j) for j in range(NP)])
        right_q = _sel(p, [P[(j + 1) % NP] for j in range(NP)])
        left_q = _sel(p, [P[(j - 1) % NP] for j in range(NP)])
        right = z * NP + right_q
        left = z * NP + left_q
        partner16 = jnp.where(z < 2, me + 2 * NP, me - 2 * NP)
        partner8 = jnp.where(lax.rem(z, 2) == 0, me + NP, me - NP)

        pending = []

        barrier_sem = pltpu.get_barrier_semaphore()
        for nbr in (left, right, partner16, partner8):
            pl.semaphore_signal(
                barrier_sem, inc=1,
                device_id=(nbr,), device_id_type=pl.DeviceIdType.MESH,
            )
        pl.semaphore_wait(barrier_sem, 4)

        x_bf = x_ref[...].astype(jnp.bfloat16)
        w_bf = w_ref[...].astype(jnp.bfloat16)
        partial_ref[...] = lax.dot(
            x_bf, w_bf, preferred_element_type=jnp.float32
        )

        def psub(c, j, lo, hi):
            return partial_ref[pl.ds(c * piece + j * subrows, subrows), lo:hi]

        def copy(src_ref, dst_ref, ssem, rsem, target):
            rdma = pltpu.make_async_remote_copy(
                src_ref=src_ref, dst_ref=dst_ref,
                send_sem=ssem, recv_sem=rsem,
                device_id=(target,), device_id_type=pl.DeviceIdType.MESH,
            )
            rdma.start()
            pending.append(rdma)
            return rdma

        abufs = (a_buf0, a_buf1)
        asems = ((a_s0, a_r0), (a_s1, a_r1))
        nbr_of = (right, left)

        def a_start(d, s, j):
            return copy(
                abufs[d].at[s, j], abufs[d].at[s + 1, j],
                asems[d][0].at[s, j], asems[d][1].at[s, j], nbr_of[d],
            )

        for j in range(NSUB):
            a_buf0[0, j, :, :] = psub(p, j, 0, nh).astype(jnp.bfloat16)
            a_buf1[0, j, :, :] = psub(p, j, nh, n).astype(jnp.bfloat16)
        arec = {}
        for d in range(2):
            for j in range(NSUB):
                arec[(d, 0, j)] = a_start(d, 0, j)
        for s in range(NP - 1):
            c0 = lax.rem(p - s - 1 + NP, NP)
            c1 = lax.rem(p + s + 1, NP)
            cs = (c0, c1)
            for j in range(NSUB):
                for d in range(2):
                    arec[(d, s, j)].wait_recv()
                    lo, hi = (0, nh) if d == 0 else (nh, n)
                    acc = (
                        abufs[d][s + 1, j, :, :].astype(jnp.float32)
                        + psub(cs[d], j, lo, hi)
                    )
                    if s < NP - 2:
                        abufs[d][s + 1, j, :, :] = acc.astype(jnp.bfloat16)
                        arec[(d, s + 1, j)] = a_start(d, s + 1, j)
                    else:
                        (own0, own1)[d][pl.ds(j * subrows, subrows), :] = acc

        r_own0 = lax.rem(p + 1, NP)
        r_own1 = lax.rem(p - 1 + NP, NP)

        half = piece // 2
        keep_off = jnp.where(z < 2, 0, half)
        send_off = jnp.where(z < 2, half, 0)
        zs1_send0[...] = own0[pl.ds(send_off, half), :].astype(jnp.bfloat16)
        e0 = copy(zs1_send0, zs1_recv0, z_s.at[0], z_r.at[0], partner16)
        zs1_send1[...] = own1[pl.ds(send_off, half), :].astype(jnp.bfloat16)
        e1 = copy(zs1_send1, zs1_recv1, z_s.at[1], z_r.at[1], partner16)
        keep2_off = z * sub
        bit0 = lax.rem(z, 2)
        send2_off = jnp.where(bit0 == 0, keep2_off + sub, keep2_off - sub)
        e0.wait_recv()
        own0[pl.ds(keep_off, half), :] = (
            own0[pl.ds(keep_off, half), :] + zs1_recv0[...].astype(jnp.float32)
        )
        zs2_send0[...] = own0[pl.ds(send2_off, sub), :].astype(jnp.bfloat16)
        f0 = copy(zs2_send0, zs2_recv0, z_s.at[2], z_r.at[2], partner8)
        e1.wait_recv()
        own1[pl.ds(keep_off, half), :] = (
            own1[pl.ds(keep_off, half), :] + zs1_recv1[...].astype(jnp.float32)
        )
        zs2_send1[...] = own1[pl.ds(send2_off, sub), :].astype(jnp.bfloat16)
        f1 = copy(zs2_send1, zs2_recv1, z_s.at[3], z_r.at[3], partner8)
        f0.wait_recv()
        fin0 = jnp.maximum(
            own0[pl.ds(keep2_off, sub), :] + zs2_recv0[...].astype(jnp.float32),
            0.0,
        )
        zg2_send0[...] = fin0.astype(jnp.bfloat16)
        g0 = copy(zg2_send0, zg2_recv0, z_s.at[4], z_r.at[4], partner8)
        f1.wait_recv()
        fin1 = jnp.maximum(
            own1[pl.ds(keep2_off, sub), :] + zs2_recv1[...].astype(jnp.float32),
            0.0,
        )
        zg2_send1[...] = fin1.astype(jnp.bfloat16)
        g1 = copy(zg2_send1, zg2_recv1, z_s.at[5], z_r.at[5], partner8)
        my_rel = bit0 * sub
        other_rel = (1 - bit0) * sub
        zg1_send0[pl.ds(my_rel, sub), :] = zg2_send0[...]
        zg1_send1[pl.ds(my_rel, sub), :] = zg2_send1[...]
        g0.wait_recv()
        zg1_send0[pl.ds(other_rel, sub), :] = zg2_recv0[...]
        h0 = copy(zg1_send0, zg1_recv0, z_s.at[6], z_r.at[6], partner16)
        g1.wait_recv()
        zg1_send1[pl.ds(other_rel, sub), :] = zg2_recv1[...]
        h1 = copy(zg1_send1, zg1_recv1, z_s.at[7], z_r.at[7], partner16)
        c_buf0[0, pl.ds(keep_off, half), :] = zg1_send0[...]
        c_buf1[0, pl.ds(keep_off, half), :] = zg1_send1[...]
        h0.wait_recv()
        c_buf0[0, pl.ds(send_off, half), :] = zg1_recv0[...]
        h1.wait_recv()
        c_buf1[0, pl.ds(send_off, half), :] = zg1_recv1[...]

        cbufs = (c_buf0, c_buf1)
        csems = ((c_s0, c_r0), (c_s1, c_r1))

        def c_start(d, h, j):
            return copy(
                cbufs[d].at[h, pl.ds(j * subrows, subrows)],
                cbufs[d].at[h + 1, pl.ds(j * subrows, subrows)],
                csems[d][0].at[h, j], csems[d][1].at[h, j], nbr_of[d],
            )

        crec = {}
        for d in range(2):
            for j in range(NSUB):
                crec[(d, 0, j)] = c_start(d, 0, j)
        out_ref[pl.ds(r_own0 * piece, piece), 0:nh] = (
            c_buf0[0, :, :].astype(jnp.float32)
        )
        out_ref[pl.ds(r_own1 * piece, piece), nh:n] = (
            c_buf1[0, :, :].astype(jnp.float32)
        )
        for h in range(NP - 1):
            g0i = lax.rem(p - h + NP, NP)
            g1i = lax.rem(p + h, NP)
            gs = (g0i, g1i)
            for j in range(NSUB):
                for d in range(2):
                    crec[(d, h, j)].wait_recv()
                    if h < NP - 2:
                        crec[(d, h + 1, j)] = c_start(d, h + 1, j)
                    lo, hi = (0, nh) if d == 0 else (nh, n)
                    out_ref[pl.ds(gs[d] * piece + j * subrows, subrows),
                            lo:hi] = (
                        cbufs[d][h + 1, pl.ds(j * subrows, subrows), :]
                        .astype(jnp.float32)
                    )

        for rdma in pending:
            rdma.wait_send()

    return pl.pallas_call(
        body,
        out_shape=jax.ShapeDtypeStruct((m, n), jnp.float32),
        in_specs=[
            pl.BlockSpec(memory_space=pltpu.VMEM),
            pl.BlockSpec(memory_space=pltpu.VMEM),
        ],
        out_specs=pl.BlockSpec(memory_space=pltpu.VMEM),
        scratch_shapes=[
            pltpu.VMEM((m, n), jnp.float32),
            pltpu.VMEM((NP, NSUB, piece // NSUB, nh), jnp.bfloat16),
            pltpu.VMEM((NP, NSUB, piece // NSUB, nh), jnp.bfloat16),
            pltpu.VMEM((NP, piece, nh), jnp.bfloat16),
            pltpu.VMEM((NP, piece, nh), jnp.bfloat16),
            pltpu.VMEM((piece, nh), jnp.float32),
            pltpu.VMEM((piece, nh), jnp.float32),
            pltpu.VMEM((piece // 2, nh), jnp.bfloat16),
            pltpu.VMEM((piece // 2, nh), jnp.bfloat16),
            pltpu.VMEM((piece // 2, nh), jnp.bfloat16),
            pltpu.VMEM((piece // 2, nh), jnp.bfloat16),
            pltpu.VMEM((sub, nh), jnp.bfloat16),
            pltpu.VMEM((sub, nh), jnp.bfloat16),
            pltpu.VMEM((sub, nh), jnp.bfloat16),
            pltpu.VMEM((sub, nh), jnp.bfloat16),
            pltpu.VMEM((sub, nh), jnp.bfloat16),
            pltpu.VMEM((sub, nh), jnp.bfloat16),
            pltpu.VMEM((sub, nh), jnp.bfloat16),
            pltpu.VMEM((sub, nh), jnp.bfloat16),
            pltpu.VMEM((piece // 2, nh), jnp.bfloat16),
            pltpu.VMEM((piece // 2, nh), jnp.bfloat16),
            pltpu.VMEM((piece // 2, nh), jnp.bfloat16),
            pltpu.VMEM((piece // 2, nh), jnp.bfloat16),
            pltpu.SemaphoreType.DMA((NP - 1, NSUB)),
            pltpu.SemaphoreType.DMA((NP - 1, NSUB)),
            pltpu.SemaphoreType.DMA((NP - 1, NSUB)),
            pltpu.SemaphoreType.DMA((NP - 1, NSUB)),
            pltpu.SemaphoreType.DMA((8,)),
            pltpu.SemaphoreType.DMA((8,)),
            pltpu.SemaphoreType.DMA((NP - 1, NSUB)),
            pltpu.SemaphoreType.DMA((NP - 1, NSUB)),
            pltpu.SemaphoreType.DMA((NP - 1, NSUB)),
            pltpu.SemaphoreType.DMA((NP - 1, NSUB)),
        ],
        compiler_params=pltpu.CompilerParams(
            collective_id=0,
            vmem_limit_bytes=100 * 1024 * 1024,
        ),
    )(x, w_mat)


# device time: 134904 ns/iter; 2.3780x vs baseline; 1.0757x over previous
import jax
import jax.numpy as jnp
from jax import lax
from jax.experimental import pallas as pl
from jax.experimental.pallas import tpu as pltpu

N_DEV = 32
NP = 8
NZ = 4
NSUB = 2
P = [0, 3, 4, 7, 6, 5, 2, 1]


def _sel(idx, table):
    out = jnp.int32(0)
    for j, v in enumerate(table):
        out = jnp.where(idx == j, jnp.int32(v), out)
    return out


def kernel(x, w_mat):
    m, k_local = x.shape
    _, n = w_mat.shape
    piece = m // NP
    subrows = piece // NSUB
    sub = piece // NZ
    nh = n // 2

    def body(x_ref, w_ref, out_ref, rel0, rel1,
             a_buf0, a_buf1, c_buf0, c_buf1, own0, own1,
             zs1_send0, zs1_recv0, zs1_send1, zs1_recv1,
             zs2_send0, zs2_recv0, zs2_send1, zs2_recv1,
             zg2_send0, zg2_recv0, zg2_send1, zg2_recv1,
             zg1_send0, zg1_recv0, zg1_send1, zg1_recv1,
             a_s0, a_r0, a_s1, a_r1, z_s, z_r, c_s0, c_r0, c_s1, c_r1):
        me = lax.axis_index("i")
        z = lax.div(me, NP)
        q = lax.rem(me, NP)
        p = _sel(q, [P.index(j) for j in range(NP)])
        right_q = _sel(p, [P[(j + 1) % NP] for j in range(NP)])
        left_q = _sel(p, [P[(j - 1) % NP] for j in range(NP)])
        right = z * NP + right_q
        left = z * NP + left_q
        partner16 = jnp.where(z < 2, me + 2 * NP, me - 2 * NP)
        partner8 = jnp.where(lax.rem(z, 2) == 0, me + NP, me - NP)

        pending = []

        barrier_sem = pltpu.get_barrier_semaphore()
        for nbr in (left, right, partner16, partner8):
            pl.semaphore_signal(
                barrier_sem, inc=1,
                device_id=(nbr,), device_id_type=pl.DeviceIdType.MESH,
            )
        pl.semaphore_wait(barrier_sem, 4)

        def copy(src_ref, dst_ref, ssem, rsem, target):
            rdma = pltpu.make_async_remote_copy(
                src_ref=src_ref, dst_ref=dst_ref,
                send_sem=ssem, recv_sem=rsem,
                device_id=(target,), device_id_type=pl.DeviceIdType.MESH,
            )
            rdma.start()
            pending.append(rdma)
            return rdma

        abufs = (a_buf0, a_buf1)
        asems = ((a_s0, a_r0), (a_s1, a_r1))
        nbr_of = (right, left)

        def a_start(d, s, j):
            return copy(
                abufs[d].at[s, j], abufs[d].at[s + 1, j],
                asems[d][0].at[s, j], asems[d][1].at[s, j], nbr_of[d],
            )

        w_bf = w_ref[...].astype(jnp.bfloat16)

        xp = x_ref[pl.ds(p * piece, piece), :].astype(jnp.bfloat16)
        pp = lax.dot(xp, w_bf, preferred_element_type=jnp.float32)
        for j in range(NSUB):
            r0_, r1_ = j * subrows, (j + 1) * subrows
            a_buf0[0, j, :, :] = pp[r0_:r1_, 0:nh].astype(jnp.bfloat16)
            a_buf1[0, j, :, :] = pp[r0_:r1_, nh:n].astype(jnp.bfloat16)
        arec = {}
        for d in range(2):
            for j in range(NSUB):
                arec[(d, 0, j)] = a_start(d, 0, j)

        for k in range(1, NP):
            ck0 = lax.rem(p - k + NP, NP)
            ck1 = lax.rem(p + k, NP)
            xk0 = x_ref[pl.ds(ck0 * piece, piece), :].astype(jnp.bfloat16)
            xk1 = x_ref[pl.ds(ck1 * piece, piece), :].astype(jnp.bfloat16)
            rel0[pl.ds(k * piece, piece), :] = lax.dot(
                xk0, w_bf[:, 0:nh], preferred_element_type=jnp.float32
            )
            rel1[pl.ds(k * piece, piece), :] = lax.dot(
                xk1, w_bf[:, nh:n], preferred_element_type=jnp.float32
            )

        rels = (rel0, rel1)

        for s in range(NP - 1):
            for j in range(NSUB):
                rows = pl.ds((s + 1) * piece + j * subrows, subrows)
                for d in range(2):
                    arec[(d, s, j)].wait_recv()
                    acc = (
                        abufs[d][s + 1, j, :, :].astype(jnp.float32)
                        + rels[d][rows, :]
                    )
                    if s < NP - 2:
                        abufs[d][s + 1, j, :, :] = acc.astype(jnp.bfloat16)
                        arec[(d, s + 1, j)] = a_start(d, s + 1, j)
                    else:
                        (own0, own1)[d][pl.ds(j * subrows, subrows), :] = acc

        r_own0 = lax.rem(p + 1, NP)
        r_own1 = lax.rem(p - 1 + NP, NP)

        half = piece // 2
        bit0 = lax.rem(z, 2)
        bit1 = lax.div(z, 2)
        keep_off = jnp.where(bit0 == 0, 0, half)
        send_off = jnp.where(bit0 == 0, half, 0)
        zs1_send0[...] = own0[pl.ds(send_off, half), :].astype(jnp.bfloat16)
        e0 = copy(zs1_send0, zs1_recv0, z_s.at[0], z_r.at[0], partner8)
        zs1_send1[...] = own1[pl.ds(send_off, half), :].astype(jnp.bfloat16)
        e1 = copy(zs1_send1, zs1_recv1, z_s.at[1], z_r.at[1], partner8)
        keep2_off = keep_off + bit1 * sub
        send2_off = keep_off + (1 - bit1) * sub
        e0.wait_recv()
        own0[pl.ds(keep_off, half), :] = (
            own0[pl.ds(keep_off, half), :] + zs1_recv0[...].astype(jnp.float32)
        )
        zs2_send0[...] = own0[pl.ds(send2_off, sub), :].astype(jnp.bfloat16)
        f0 = copy(zs2_send0, zs2_recv0, z_s.at[2], z_r.at[2], partner16)
        e1.wait_recv()
        own1[pl.ds(keep_off, half), :] = (
            own1[pl.ds(keep_off, half), :] + zs1_recv1[...].astype(jnp.float32)
        )
        zs2_send1[...] = own1[pl.ds(send2_off, sub), :].astype(jnp.bfloat16)
        f1 = copy(zs2_send1, zs2_recv1, z_s.at[3], z_r.at[3], partner16)
        f0.wait_recv()
        fin0 = jnp.maximum(
            own0[pl.ds(keep2_off, sub), :] + zs2_recv0[...].astype(jnp.float32),
            0.0,
        )
        zg2_send0[...] = fin0.astype(jnp.bfloat16)
        g0 = copy(zg2_send0, zg2_recv0, z_s.at[4], z_r.at[4], partner16)
        f1.wait_recv()
        fin1 = jnp.maximum(
            own1[pl.ds(keep2_off, sub), :] + zs2_recv1[...].astype(jnp.float32),
            0.0,
        )
        zg2_send1[...] = fin1.astype(jnp.bfloat16)
        g1 = copy(zg2_send1, zg2_recv1, z_s.at[5], z_r.at[5], partner16)
        my_rel = bit1 * sub
        other_rel = (1 - bit1) * sub
        zg1_send0[pl.ds(my_rel, sub), :] = zg2_send0[...]
        zg1_send1[pl.ds(my_rel, sub), :] = zg2_send1[...]
        g0.wait_recv()
        zg1_send0[pl.ds(other_rel, sub), :] = zg2_recv0[...]
        h0 = copy(zg1_send0, zg1_recv0, z_s.at[6], z_r.at[6], partner8)
        g1.wait_recv()
        zg1_send1[pl.ds(other_rel, sub), :] = zg2_recv1[...]
        h1 = copy(zg1_send1, zg1_recv1, z_s.at[7], z_r.at[7], partner8)
        c_buf0[0, pl.ds(keep_off, half), :] = zg1_send0[...]
        c_buf1[0, pl.ds(keep_off, half), :] = zg1_send1[...]
        h0.wait_recv()
        c_buf0[0, pl.ds(send_off, half), :] = zg1_recv0[...]
        h1.wait_recv()
        c_buf1[0, pl.ds(send_off, half), :] = zg1_recv1[...]

        cbufs = (c_buf0, c_buf1)
        csems = ((c_s0, c_r0), (c_s1, c_r1))

        def c_start(d, h, j):
            return copy(
                cbufs[d].at[h, pl.ds(j * subrows, subrows)],
                cbufs[d].at[h + 1, pl.ds(j * subrows, subrows)],
                csems[d][0].at[h, j], csems[d][1].at[h, j], nbr_of[d],
            )

        crec = {}
        for d in range(2):
            for j in range(NSUB):
                crec[(d, 0, j)] = c_start(d, 0, j)
        out_ref[pl.ds(r_own0 * piece, piece), 0:nh] = (
            c_buf0[0, :, :].astype(jnp.float32)
        )
        out_ref[pl.ds(r_own1 * piece, piece), nh:n] = (
            c_buf1[0, :, :].astype(jnp.float32)
        )
        for h in range(NP - 1):
            g0i = lax.rem(p - h + NP, NP)
            g1i = lax.rem(p + h, NP)
            gs = (g0i, g1i)
            for j in range(NSUB):
                for d in range(2):
                    crec[(d, h, j)].wait_recv()
                    if h < NP - 2:
                        crec[(d, h + 1, j)] = c_start(d, h + 1, j)
                    lo, hi = (0, nh) if d == 0 else (nh, n)
                    out_ref[pl.ds(gs[d] * piece + j * subrows, subrows),
                            lo:hi] = (
                        cbufs[d][h + 1, pl.ds(j * subrows, subrows), :]
                        .astype(jnp.float32)
                    )

        for rdma in pending:
            rdma.wait_send()

    return pl.pallas_call(
        body,
        out_shape=jax.ShapeDtypeStruct((m, n), jnp.float32),
        in_specs=[
            pl.BlockSpec(memory_space=pltpu.VMEM),
            pl.BlockSpec(memory_space=pltpu.VMEM),
        ],
        out_specs=pl.BlockSpec(memory_space=pltpu.VMEM),
        scratch_shapes=[
            pltpu.VMEM((m, nh), jnp.float32),
            pltpu.VMEM((m, nh), jnp.float32),
            pltpu.VMEM((NP, NSUB, piece // NSUB, nh), jnp.bfloat16),
            pltpu.VMEM((NP, NSUB, piece // NSUB, nh), jnp.bfloat16),
            pltpu.VMEM((NP, piece, nh), jnp.bfloat16),
            pltpu.VMEM((NP, piece, nh), jnp.bfloat16),
            pltpu.VMEM((piece, nh), jnp.float32),
            pltpu.VMEM((piece, nh), jnp.float32),
            pltpu.VMEM((piece // 2, nh), jnp.bfloat16),
            pltpu.VMEM((piece // 2, nh), jnp.bfloat16),
            pltpu.VMEM((piece // 2, nh), jnp.bfloat16),
            pltpu.VMEM((piece // 2, nh), jnp.bfloat16),
            pltpu.VMEM((sub, nh), jnp.bfloat16),
            pltpu.VMEM((sub, nh), jnp.bfloat16),
            pltpu.VMEM((sub, nh), jnp.bfloat16),
            pltpu.VMEM((sub, nh), jnp.bfloat16),
            pltpu.VMEM((sub, nh), jnp.bfloat16),
            pltpu.VMEM((sub, nh), jnp.bfloat16),
            pltpu.VMEM((sub, nh), jnp.bfloat16),
            pltpu.VMEM((sub, nh), jnp.bfloat16),
            pltpu.VMEM((piece // 2, nh), jnp.bfloat16),
            pltpu.VMEM((piece // 2, nh), jnp.bfloat16),
            pltpu.VMEM((piece // 2, nh), jnp.bfloat16),
            pltpu.VMEM((piece // 2, nh), jnp.bfloat16),
            pltpu.SemaphoreType.DMA((NP - 1, NSUB)),
            pltpu.SemaphoreType.DMA((NP - 1, NSUB)),
            pltpu.SemaphoreType.DMA((NP - 1, NSUB)),
            pltpu.SemaphoreType.DMA((NP - 1, NSUB)),
            pltpu.SemaphoreType.DMA((8,)),
            pltpu.SemaphoreType.DMA((8,)),
            pltpu.SemaphoreType.DMA((NP - 1, NSUB)),
            pltpu.SemaphoreType.DMA((NP - 1, NSUB)),
            pltpu.SemaphoreType.DMA((NP - 1, NSUB)),
            pltpu.SemaphoreType.DMA((NP - 1, NSUB)),
        ],
        compiler_params=pltpu.CompilerParams(
            collective_id=0,
            vmem_limit_bytes=100 * 1024 * 1024,
        ),
    )(x, w_mat)


# device time: 128366 ns/iter; 2.4991x vs baseline; 1.0509x over previous
import jax
import jax.numpy as jnp
from jax import lax
from jax.experimental import pallas as pl
from jax.experimental.pallas import tpu as pltpu

N_DEV = 32
NP = 8
NZ = 4
NSUB = 2
P = [0, 3, 4, 7, 6, 5, 2, 1]


def _sel(idx, table):
    out = jnp.int32(0)
    for j, v in enumerate(table):
        out = jnp.where(idx == j, jnp.int32(v), out)
    return out


def kernel(x, w_mat):
    m, k_local = x.shape
    _, n = w_mat.shape
    piece = m // NP
    subrows = piece // NSUB
    sub = piece // NZ
    nh = n // 2

    def body(x_ref, w_ref, out_ref, rel0, rel1,
             a_buf0, a_buf1, c_buf0, c_buf1, own0, own1,
             zs1_send0, zs1_recv0, zs1_send1, zs1_recv1,
             zs2_send0, zs2_recv0, zs2_send1, zs2_recv1,
             zg2_send0, zg2_recv0, zg2_send1, zg2_recv1,
             zg1_send0, zg1_recv0, zg1_send1, zg1_recv1,
             a_s0, a_r0, a_s1, a_r1, z_s, z_r, c_s0, c_r0, c_s1, c_r1):
        me = lax.axis_index("i")
        z = lax.div(me, NP)
        q = lax.rem(me, NP)
        p = _sel(q, [P.index(j) for j in range(NP)])
        right_q = _sel(p, [P[(j + 1) % NP] for j in range(NP)])
        left_q = _sel(p, [P[(j - 1) % NP] for j in range(NP)])
        right = z * NP + right_q
        left = z * NP + left_q
        partner16 = jnp.where(z < 2, me + 2 * NP, me - 2 * NP)
        partner8 = jnp.where(lax.rem(z, 2) == 0, me + NP, me - NP)

        pending = []

        barrier_sem = pltpu.get_barrier_semaphore()
        for nbr in (left, right, partner16, partner8):
            pl.semaphore_signal(
                barrier_sem, inc=1,
                device_id=(nbr,), device_id_type=pl.DeviceIdType.MESH,
            )
        pl.semaphore_wait(barrier_sem, 4)

        def copy(src_ref, dst_ref, ssem, rsem, target):
            rdma = pltpu.make_async_remote_copy(
                src_ref=src_ref, dst_ref=dst_ref,
                send_sem=ssem, recv_sem=rsem,
                device_id=(target,), device_id_type=pl.DeviceIdType.MESH,
            )
            rdma.start()
            pending.append(rdma)
            return rdma

        abufs = (a_buf0, a_buf1)
        asems = ((a_s0, a_r0), (a_s1, a_r1))
        nbr_of = (right, left)

        def a_start(d, s, j):
            return copy(
                abufs[d].at[s, j], abufs[d].at[s + 1, j],
                asems[d][0].at[s, j], asems[d][1].at[s, j], nbr_of[d],
            )

        w_bf = w_ref[...].astype(jnp.bfloat16)

        half = piece // 2
        bit0 = lax.rem(z, 2)
        bit1 = lax.div(z, 2)
        keep_off = jnp.where(bit0 == 0, 0, half)
        send_off = jnp.where(bit0 == 0, half, 0)
        r_sub = (send_off, keep_off)

        xp = x_ref[pl.ds(p * piece, piece), :].astype(jnp.bfloat16)
        pp = lax.dot(xp, w_bf, preferred_element_type=jnp.float32)
        rel0[0:piece, :] = pp[:, 0:nh]
        rel1[0:piece, :] = pp[:, nh:n]
        arec = {}
        for j in range(NSUB):
            rows = pl.ds(r_sub[j], subrows)
            a_buf0[0, j, :, :] = rel0[rows, :].astype(jnp.bfloat16)
            a_buf1[0, j, :, :] = rel1[rows, :].astype(jnp.bfloat16)
            for d in range(2):
                arec[(d, 0, j)] = a_start(d, 0, j)

        for k in range(1, NP):
            ck0 = lax.rem(p - k + NP, NP)
            ck1 = lax.rem(p + k, NP)
            xk0 = x_ref[pl.ds(ck0 * piece, piece), :].astype(jnp.bfloat16)
            xk1 = x_ref[pl.ds(ck1 * piece, piece), :].astype(jnp.bfloat16)
            rel0[pl.ds(k * piece, piece), :] = lax.dot(
                xk0, w_bf[:, 0:nh], preferred_element_type=jnp.float32
            )
            rel1[pl.ds(k * piece, piece), :] = lax.dot(
                xk1, w_bf[:, nh:n], preferred_element_type=jnp.float32
            )

        rels = (rel0, rel1)

        zs1 = {}
        for s in range(NP - 1):
            for j in range(NSUB):
                rows = pl.ds((s + 1) * piece + r_sub[j], subrows)
                for d in range(2):
                    arec[(d, s, j)].wait_recv()
                    acc = (
                        abufs[d][s + 1, j, :, :].astype(jnp.float32)
                        + rels[d][rows, :]
                    )
                    if s < NP - 2:
                        abufs[d][s + 1, j, :, :] = acc.astype(jnp.bfloat16)
                        arec[(d, s + 1, j)] = a_start(d, s + 1, j)
                    else:
                        (own0, own1)[d][pl.ds(r_sub[j], subrows), :] = acc
                        if j == 0:
                            zsend = (zs1_send0, zs1_send1)[d]
                            zrecv = (zs1_recv0, zs1_recv1)[d]
                            zsend[...] = acc.astype(jnp.bfloat16)
                            zs1[d] = copy(zsend, zrecv, z_s.at[d],
                                          z_r.at[d], partner8)

        r_own0 = lax.rem(p + 1, NP)
        r_own1 = lax.rem(p - 1 + NP, NP)

        e0, e1 = zs1[0], zs1[1]
        keep2_off = keep_off + bit1 * sub
        send2_off = keep_off + (1 - bit1) * sub
        e0.wait_recv()
        own0[pl.ds(keep_off, half), :] = (
            own0[pl.ds(keep_off, half), :] + zs1_recv0[...].astype(jnp.float32)
        )
        zs2_send0[...] = own0[pl.ds(send2_off, sub), :].astype(jnp.bfloat16)
        f0 = copy(zs2_send0, zs2_recv0, z_s.at[2], z_r.at[2], partner16)
        e1.wait_recv()
        own1[pl.ds(keep_off, half), :] = (
            own1[pl.ds(keep_off, half), :] + zs1_recv1[...].astype(jnp.float32)
        )
        zs2_send1[...] = own1[pl.ds(send2_off, sub), :].astype(jnp.bfloat16)
        f1 = copy(zs2_send1, zs2_recv1, z_s.at[3], z_r.at[3], partner16)
        f0.wait_recv()
        fin0 = jnp.maximum(
            own0[pl.ds(keep2_off, sub), :] + zs2_recv0[...].astype(jnp.float32),
            0.0,
        )
        zg2_send0[...] = fin0.astype(jnp.bfloat16)
        g0 = copy(zg2_send0, zg2_recv0, z_s.at[4], z_r.at[4], partner16)
        f1.wait_recv()
        fin1 = jnp.maximum(
            own1[pl.ds(keep2_off, sub), :] + zs2_recv1[...].astype(jnp.float32),
            0.0,
        )
        zg2_send1[...] = fin1.astype(jnp.bfloat16)
        g1 = copy(zg2_send1, zg2_recv1, z_s.at[5], z_r.at[5], partner16)
        cbufs = (c_buf0, c_buf1)
        csems = ((c_s0, c_r0), (c_s1, c_r1))
        c_rows = (keep_off, send_off)

        def c_start(d, h, j):
            return copy(
                cbufs[d].at[h, pl.ds(c_rows[j], subrows)],
                cbufs[d].at[h + 1, pl.ds(c_rows[j], subrows)],
                csems[d][0].at[h, j], csems[d][1].at[h, j], nbr_of[d],
            )

        crec = {}
        my_rel = bit1 * sub
        other_rel = (1 - bit1) * sub
        zg1_send0[pl.ds(my_rel, sub), :] = zg2_send0[...]
        zg1_send1[pl.ds(my_rel, sub), :] = zg2_send1[...]
        g0.wait_recv()
        zg1_send0[pl.ds(other_rel, sub), :] = zg2_recv0[...]
        h0 = copy(zg1_send0, zg1_recv0, z_s.at[6], z_r.at[6], partner8)
        c_buf0[0, pl.ds(keep_off, half), :] = zg1_send0[...]
        crec[(0, 0, 0)] = c_start(0, 0, 0)
        g1.wait_recv()
        zg1_send1[pl.ds(other_rel, sub), :] = zg2_recv1[...]
        h1 = copy(zg1_send1, zg1_recv1, z_s.at[7], z_r.at[7], partner8)
        c_buf1[0, pl.ds(keep_off, half), :] = zg1_send1[...]
        crec[(1, 0, 0)] = c_start(1, 0, 0)
        h0.wait_recv()
        c_buf0[0, pl.ds(send_off, half), :] = zg1_recv0[...]
        crec[(0, 0, 1)] = c_start(0, 0, 1)
        h1.wait_recv()
        c_buf1[0, pl.ds(send_off, half), :] = zg1_recv1[...]
        crec[(1, 0, 1)] = c_start(1, 0, 1)
        out_ref[pl.ds(r_own0 * piece, piece), 0:nh] = (
            c_buf0[0, :, :].astype(jnp.float32)
        )
        out_ref[pl.ds(r_own1 * piece, piece), nh:n] = (
            c_buf1[0, :, :].astype(jnp.float32)
        )

        for h in range(NP - 1):
            g0i = lax.rem(p - h + NP, NP)
            g1i = lax.rem(p + h, NP)
            gs = (g0i, g1i)
            for j in range(NSUB):
                for d in range(2):
                    crec[(d, h, j)].wait_recv()
                    if h < NP - 2:
                        crec[(d, h + 1, j)] = c_start(d, h + 1, j)
                    lo, hi = (0, nh) if d == 0 else (nh, n)
                    out_ref[pl.ds(gs[d] * piece + c_rows[j], subrows),
                            lo:hi] = (
                        cbufs[d][h + 1, pl.ds(c_rows[j], subrows), :]
                        .astype(jnp.float32)
                    )

        for rdma in pending:
            rdma.wait_send()

    return pl.pallas_call(
        body,
        out_shape=jax.ShapeDtypeStruct((m, n), jnp.float32),
        in_specs=[
            pl.BlockSpec(memory_space=pltpu.VMEM),
            pl.BlockSpec(memory_space=pltpu.VMEM),
        ],
        out_specs=pl.BlockSpec(memory_space=pltpu.VMEM),
        scratch_shapes=[
            pltpu.VMEM((m, nh), jnp.float32),
            pltpu.VMEM((m, nh), jnp.float32),
            pltpu.VMEM((NP, NSUB, piece // NSUB, nh), jnp.bfloat16),
            pltpu.VMEM((NP, NSUB, piece // NSUB, nh), jnp.bfloat16),
            pltpu.VMEM((NP, piece, nh), jnp.bfloat16),
            pltpu.VMEM((NP, piece, nh), jnp.bfloat16),
            pltpu.VMEM((piece, nh), jnp.float32),
            pltpu.VMEM((piece, nh), jnp.float32),
            pltpu.VMEM((piece // 2, nh), jnp.bfloat16),
            pltpu.VMEM((piece // 2, nh), jnp.bfloat16),
            pltpu.VMEM((piece // 2, nh), jnp.bfloat16),
            pltpu.VMEM((piece // 2, nh), jnp.bfloat16),
            pltpu.VMEM((sub, nh), jnp.bfloat16),
            pltpu.VMEM((sub, nh), jnp.bfloat16),
            pltpu.VMEM((sub, nh), jnp.bfloat16),
            pltpu.VMEM((sub, nh), jnp.bfloat16),
            pltpu.VMEM((sub, nh), jnp.bfloat16),
            pltpu.VMEM((sub, nh), jnp.bfloat16),
            pltpu.VMEM((sub, nh), jnp.bfloat16),
            pltpu.VMEM((sub, nh), jnp.bfloat16),
            pltpu.VMEM((piece // 2, nh), jnp.bfloat16),
            pltpu.VMEM((piece // 2, nh), jnp.bfloat16),
            pltpu.VMEM((piece // 2, nh), jnp.bfloat16),
            pltpu.VMEM((piece // 2, nh), jnp.bfloat16),
            pltpu.SemaphoreType.DMA((NP - 1, NSUB)),
            pltpu.SemaphoreType.DMA((NP - 1, NSUB)),
            pltpu.SemaphoreType.DMA((NP - 1, NSUB)),
            pltpu.SemaphoreType.DMA((NP - 1, NSUB)),
            pltpu.SemaphoreType.DMA((8,)),
            pltpu.SemaphoreType.DMA((8,)),
            pltpu.SemaphoreType.DMA((NP - 1, NSUB)),
            pltpu.SemaphoreType.DMA((NP - 1, NSUB)),
            pltpu.SemaphoreType.DMA((NP - 1, NSUB)),
            pltpu.SemaphoreType.DMA((NP - 1, NSUB)),
        ],
        compiler_params=pltpu.CompilerParams(
            collective_id=0,
            vmem_limit_bytes=100 * 1024 * 1024,
        ),
    )(x, w_mat)


# device time: 126772 ns/iter; 2.5305x vs baseline; 1.0126x over previous
import jax
import jax.numpy as jnp
from jax import lax
from jax.experimental import pallas as pl
from jax.experimental.pallas import tpu as pltpu

N_DEV = 32
NP = 8
NZ = 4
NSUB = 2
P = [0, 3, 4, 7, 6, 5, 2, 1]


def _sel(idx, table):
    out = jnp.int32(0)
    for j, v in enumerate(table):
        out = jnp.where(idx == j, jnp.int32(v), out)
    return out


def kernel(x, w_mat):
    m, k_local = x.shape
    _, n = w_mat.shape
    piece = m // NP
    subrows = piece // NSUB
    sub = piece // NZ
    nh = n // 2

    def body(x_ref, w_ref, out_ref, rel0, rel1,
             a_buf0, a_buf1, c_buf0, c_buf1, own0, own1,
             zs1_send0, zs1_recv0, zs1_send1, zs1_recv1,
             zx_send0, zx_recv0, zx_send1, zx_recv1,
             zg1_send0, zg1_recv0, zg1_send1, zg1_recv1,
             a_s0, a_r0, a_s1, a_r1, z_s, z_r, c_s0, c_r0, c_s1, c_r1):
        me = lax.axis_index("i")
        z = lax.div(me, NP)
        q = lax.rem(me, NP)
        p = _sel(q, [P.index(j) for j in range(NP)])
        right_q = _sel(p, [P[(j + 1) % NP] for j in range(NP)])
        left_q = _sel(p, [P[(j - 1) % NP] for j in range(NP)])
        right = z * NP + right_q
        left = z * NP + left_q
        partner16 = jnp.where(z < 2, me + 2 * NP, me - 2 * NP)
        partner8 = jnp.where(lax.rem(z, 2) == 0, me + NP, me - NP)

        pending = []

        barrier_sem = pltpu.get_barrier_semaphore()
        for nbr in (left, right, partner16, partner8):
            pl.semaphore_signal(
                barrier_sem, inc=1,
                device_id=(nbr,), device_id_type=pl.DeviceIdType.MESH,
            )
        pl.semaphore_wait(barrier_sem, 4)

        def copy(src_ref, dst_ref, ssem, rsem, target):
            rdma = pltpu.make_async_remote_copy(
                src_ref=src_ref, dst_ref=dst_ref,
                send_sem=ssem, recv_sem=rsem,
                device_id=(target,), device_id_type=pl.DeviceIdType.MESH,
            )
            rdma.start()
            pending.append(rdma)
            return rdma

        abufs = (a_buf0, a_buf1)
        asems = ((a_s0, a_r0), (a_s1, a_r1))
        nbr_of = (right, left)

        def a_start(d, s, j):
            return copy(
                abufs[d].at[s, j], abufs[d].at[s + 1, j],
                asems[d][0].at[s, j], asems[d][1].at[s, j], nbr_of[d],
            )

        w_bf = w_ref[...].astype(jnp.bfloat16)

        half = piece // 2
        bit0 = lax.rem(z, 2)
        keep_off = jnp.where(bit0 == 0, 0, half)
        send_off = jnp.where(bit0 == 0, half, 0)
        r_sub = (send_off, keep_off)

        xp = x_ref[pl.ds(p * piece, piece), :].astype(jnp.bfloat16)
        pp = lax.dot(xp, w_bf, preferred_element_type=jnp.float32)
        rel0[0:piece, :] = pp[:, 0:nh]
        rel1[0:piece, :] = pp[:, nh:n]
        arec = {}
        for j in range(NSUB):
            rows = pl.ds(r_sub[j], subrows)
            a_buf0[0, j, :, :] = rel0[rows, :].astype(jnp.bfloat16)
            a_buf1[0, j, :, :] = rel1[rows, :].astype(jnp.bfloat16)
            for d in range(2):
                arec[(d, 0, j)] = a_start(d, 0, j)

        for k in range(1, NP):
            ck0 = lax.rem(p - k + NP, NP)
            ck1 = lax.rem(p + k, NP)
            xk0 = x_ref[pl.ds(ck0 * piece, piece), :].astype(jnp.bfloat16)
            xk1 = x_ref[pl.ds(ck1 * piece, piece), :].astype(jnp.bfloat16)
            rel0[pl.ds(k * piece, piece), :] = lax.dot(
                xk0, w_bf[:, 0:nh], preferred_element_type=jnp.float32
            )
            rel1[pl.ds(k * piece, piece), :] = lax.dot(
                xk1, w_bf[:, nh:n], preferred_element_type=jnp.float32
            )

        rels = (rel0, rel1)

        zs1 = {}
        for s in range(NP - 1):
            for j in range(NSUB):
                rows = pl.ds((s + 1) * piece + r_sub[j], subrows)
                for d in range(2):
                    arec[(d, s, j)].wait_recv()
                    acc = (
                        abufs[d][s + 1, j, :, :].astype(jnp.float32)
                        + rels[d][rows, :]
                    )
                    if s < NP - 2:
                        abufs[d][s + 1, j, :, :] = acc.astype(jnp.bfloat16)
                        arec[(d, s + 1, j)] = a_start(d, s + 1, j)
                    else:
                        (own0, own1)[d][pl.ds(r_sub[j], subrows), :] = acc
                        if j == 0:
                            zsend = (zs1_send0, zs1_send1)[d]
                            zrecv = (zs1_recv0, zs1_recv1)[d]
                            zsend[...] = acc.astype(jnp.bfloat16)
                            zs1[d] = copy(zsend, zrecv, z_s.at[d],
                                          z_r.at[d], partner8)

        r_own0 = lax.rem(p + 1, NP)
        r_own1 = lax.rem(p - 1 + NP, NP)

        e0, e1 = zs1[0], zs1[1]
        e0.wait_recv()
        own0[pl.ds(keep_off, half), :] = (
            own0[pl.ds(keep_off, half), :] + zs1_recv0[...].astype(jnp.float32)
        )
        zx_send0[...] = own0[pl.ds(keep_off, half), :].astype(jnp.bfloat16)
        f0 = copy(zx_send0, zx_recv0, z_s.at[2], z_r.at[2], partner16)
        e1.wait_recv()
        own1[pl.ds(keep_off, half), :] = (
            own1[pl.ds(keep_off, half), :] + zs1_recv1[...].astype(jnp.float32)
        )
        zx_send1[...] = own1[pl.ds(keep_off, half), :].astype(jnp.bfloat16)
        f1 = copy(zx_send1, zx_recv1, z_s.at[3], z_r.at[3], partner16)
        cbufs = (c_buf0, c_buf1)
        csems = ((c_s0, c_r0), (c_s1, c_r1))
        c_rows = (keep_off, send_off)
        zg1s = ((zg1_send0, zg1_recv0), (zg1_send1, zg1_recv1))

        def c_start(d, h, j):
            src = zg1s[d][j] if h == 0 else (
                cbufs[d].at[h, pl.ds(c_rows[j], subrows)]
            )
            return copy(
                src,
                cbufs[d].at[h + 1, pl.ds(c_rows[j], subrows)],
                csems[d][0].at[h, j], csems[d][1].at[h, j], nbr_of[d],
            )

        crec = {}
        f0.wait_recv()
        finh0 = jnp.maximum(
            own0[pl.ds(keep_off, half), :] + zx_recv0[...].astype(jnp.float32),
            0.0,
        )
        zg1_send0[...] = finh0.astype(jnp.bfloat16)
        h0 = copy(zg1_send0, zg1_recv0, z_s.at[4], z_r.at[4], partner8)
        crec[(0, 0, 0)] = c_start(0, 0, 0)
        f1.wait_recv()
        finh1 = jnp.maximum(
            own1[pl.ds(keep_off, half), :] + zx_recv1[...].astype(jnp.float32),
            0.0,
        )
        zg1_send1[...] = finh1.astype(jnp.bfloat16)
        h1 = copy(zg1_send1, zg1_recv1, z_s.at[5], z_r.at[5], partner8)
        crec[(1, 0, 0)] = c_start(1, 0, 0)
        out_ref[pl.ds(r_own0 * piece + keep_off, half), 0:nh] = (
            zg1_send0[...].astype(jnp.float32)
        )
        out_ref[pl.ds(r_own1 * piece + keep_off, half), nh:n] = (
            zg1_send1[...].astype(jnp.float32)
        )
        h0.wait_recv()
        crec[(0, 0, 1)] = c_start(0, 0, 1)
        h1.wait_recv()
        crec[(1, 0, 1)] = c_start(1, 0, 1)
        out_ref[pl.ds(r_own0 * piece + send_off, half), 0:nh] = (
            zg1_recv0[...].astype(jnp.float32)
        )
        out_ref[pl.ds(r_own1 * piece + send_off, half), nh:n] = (
            zg1_recv1[...].astype(jnp.float32)
        )

        for h in range(NP - 1):
            g0i = lax.rem(p - h + NP, NP)
            g1i = lax.rem(p + h, NP)
            gs = (g0i, g1i)
            for j in range(NSUB):
                for d in range(2):
                    crec[(d, h, j)].wait_recv()
                    if h < NP - 2:
                        crec[(d, h + 1, j)] = c_start(d, h + 1, j)
                    lo, hi = (0, nh) if d == 0 else (nh, n)
                    out_ref[pl.ds(gs[d] * piece + c_rows[j], subrows),
                            lo:hi] = (
                        cbufs[d][h + 1, pl.ds(c_rows[j], subrows), :]
                        .astype(jnp.float32)
                    )

        for rdma in pending:
            rdma.wait_send()

    return pl.pallas_call(
        body,
        out_shape=jax.ShapeDtypeStruct((m, n), jnp.float32),
        in_specs=[
            pl.BlockSpec(memory_space=pltpu.VMEM),
            pl.BlockSpec(memory_space=pltpu.VMEM),
        ],
        out_specs=pl.BlockSpec(memory_space=pltpu.VMEM),
        scratch_shapes=[
            pltpu.VMEM((m, nh), jnp.float32),
            pltpu.VMEM((m, nh), jnp.float32),
            pltpu.VMEM((NP, NSUB, piece // NSUB, nh), jnp.bfloat16),
            pltpu.VMEM((NP, NSUB, piece // NSUB, nh), jnp.bfloat16),
            pltpu.VMEM((NP, piece, nh), jnp.bfloat16),
            pltpu.VMEM((NP, piece, nh), jnp.bfloat16),
            pltpu.VMEM((piece, nh), jnp.float32),
            pltpu.VMEM((piece, nh), jnp.float32),
            pltpu.VMEM((piece // 2, nh), jnp.bfloat16),
            pltpu.VMEM((piece // 2, nh), jnp.bfloat16),
            pltpu.VMEM((piece // 2, nh), jnp.bfloat16),
            pltpu.VMEM((piece // 2, nh), jnp.bfloat16),
            pltpu.VMEM((piece // 2, nh), jnp.bfloat16),
            pltpu.VMEM((piece // 2, nh), jnp.bfloat16),
            pltpu.VMEM((piece // 2, nh), jnp.bfloat16),
            pltpu.VMEM((piece // 2, nh), jnp.bfloat16),
            pltpu.VMEM((piece // 2, nh), jnp.bfloat16),
            pltpu.VMEM((piece // 2, nh), jnp.bfloat16),
            pltpu.VMEM((piece // 2, nh), jnp.bfloat16),
            pltpu.VMEM((piece // 2, nh), jnp.bfloat16),
            pltpu.SemaphoreType.DMA((NP - 1, NSUB)),
            pltpu.SemaphoreType.DMA((NP - 1, NSUB)),
            pltpu.SemaphoreType.DMA((NP - 1, NSUB)),
            pltpu.SemaphoreType.DMA((NP - 1, NSUB)),
            pltpu.SemaphoreType.DMA((8,)),
            pltpu.SemaphoreType.DMA((8,)),
            pltpu.SemaphoreType.DMA((NP - 1, NSUB)),
            pltpu.SemaphoreType.DMA((NP - 1, NSUB)),
            pltpu.SemaphoreType.DMA((NP - 1, NSUB)),
            pltpu.SemaphoreType.DMA((NP - 1, NSUB)),
        ],
        compiler_params=pltpu.CompilerParams(
            collective_id=0,
            vmem_limit_bytes=100 * 1024 * 1024,
        ),
    )(x, w_mat)


# device time: 126509 ns/iter; 2.5358x vs baseline; 1.0021x over previous
import jax
import jax.numpy as jnp
from jax import lax
from jax.experimental import pallas as pl
from jax.experimental.pallas import tpu as pltpu

N_DEV = 32
NP = 8
NZ = 4
NSUB = 2
P = [0, 3, 4, 7, 6, 5, 2, 1]


def _sel(idx, table):
    out = jnp.int32(0)
    for j, v in enumerate(table):
        out = jnp.where(idx == j, jnp.int32(v), out)
    return out


def kernel(x, w_mat):
    m, k_local = x.shape
    _, n = w_mat.shape
    piece = m // NP
    subrows = piece // NSUB
    sub = piece // NZ
    nh = n // 2

    def body(x_ref, w_ref, out_ref, rel0, rel1,
             a_buf0, a_buf1, c_buf0, c_buf1, own0, own1,
             zs1_send0, zs1_recv0, zs1_send1, zs1_recv1,
             zx_send0, zx_recv0, zx_send1, zx_recv1,
             zg1_send0, zg1_recv0, zg1_send1, zg1_recv1,
             a_s0, a_r0, a_s1, a_r1, z_s, z_r, c_s0, c_r0, c_s1, c_r1):
        me = lax.axis_index("i")
        z = lax.div(me, NP)
        q = lax.rem(me, NP)
        p = _sel(q, [P.index(j) for j in range(NP)])
        right_q = _sel(p, [P[(j + 1) % NP] for j in range(NP)])
        left_q = _sel(p, [P[(j - 1) % NP] for j in range(NP)])
        right = z * NP + right_q
        left = z * NP + left_q
        partner16 = jnp.where(z < 2, me + 2 * NP, me - 2 * NP)
        partner8 = jnp.where(lax.rem(z, 2) == 0, me + NP, me - NP)

        pending = []

        barrier_sem = pltpu.get_barrier_semaphore()
        for nbr in (left, right, partner16, partner8):
            pl.semaphore_signal(
                barrier_sem, inc=1,
                device_id=(nbr,), device_id_type=pl.DeviceIdType.MESH,
            )
        pl.semaphore_wait(barrier_sem, 4)

        def copy(src_ref, dst_ref, ssem, rsem, target):
            rdma = pltpu.make_async_remote_copy(
                src_ref=src_ref, dst_ref=dst_ref,
                send_sem=ssem, recv_sem=rsem,
                device_id=(target,), device_id_type=pl.DeviceIdType.MESH,
            )
            rdma.start()
            pending.append(rdma)
            return rdma

        abufs = (a_buf0, a_buf1)
        asems = ((a_s0, a_r0), (a_s1, a_r1))
        nbr_of = (right, left)

        def a_start(d, s, j):
            return copy(
                abufs[d].at[s, j], abufs[d].at[s + 1, j],
                asems[d][0].at[s, j], asems[d][1].at[s, j], nbr_of[d],
            )

        w_bf = w_ref[...].astype(jnp.bfloat16)

        half = piece // 2
        bit0 = lax.rem(z, 2)
        keep_off = jnp.where(bit0 == 0, 0, half)
        send_off = jnp.where(bit0 == 0, half, 0)
        r_sub = (send_off, keep_off)

        arec = {}
        for j in range(NSUB):
            xj = x_ref[pl.ds(p * piece + r_sub[j], subrows), :].astype(
                jnp.bfloat16
            )
            a_buf0[0, j, :, :] = lax.dot(
                xj, w_bf[:, 0:nh], preferred_element_type=jnp.float32
            ).astype(jnp.bfloat16)
            a_buf1[0, j, :, :] = lax.dot(
                xj, w_bf[:, nh:n], preferred_element_type=jnp.float32
            ).astype(jnp.bfloat16)
            for d in range(2):
                arec[(d, 0, j)] = a_start(d, 0, j)

        for k in range(1, NP):
            ck0 = lax.rem(p - k + NP, NP)
            ck1 = lax.rem(p + k, NP)
            xk0 = x_ref[pl.ds(ck0 * piece, piece), :].astype(jnp.bfloat16)
            xk1 = x_ref[pl.ds(ck1 * piece, piece), :].astype(jnp.bfloat16)
            rel0[pl.ds(k * piece, piece), :] = lax.dot(
                xk0, w_bf[:, 0:nh], preferred_element_type=jnp.float32
            )
            rel1[pl.ds(k * piece, piece), :] = lax.dot(
                xk1, w_bf[:, nh:n], preferred_element_type=jnp.float32
            )

        rels = (rel0, rel1)

        zs1 = {}
        for s in range(NP - 1):
            for j in range(NSUB):
                rows = pl.ds((s + 1) * piece + r_sub[j], subrows)
                for d in range(2):
                    arec[(d, s, j)].wait_recv()
                    acc = (
                        abufs[d][s + 1, j, :, :].astype(jnp.float32)
                        + rels[d][rows, :]
                    )
                    if s < NP - 2:
                        abufs[d][s + 1, j, :, :] = acc.astype(jnp.bfloat16)
                        arec[(d, s + 1, j)] = a_start(d, s + 1, j)
                    else:
                        (own0, own1)[d][pl.ds(r_sub[j], subrows), :] = acc
                        if j == 0:
                            zsend = (zs1_send0, zs1_send1)[d]
                            zrecv = (zs1_recv0, zs1_recv1)[d]
                            zsend[...] = acc.astype(jnp.bfloat16)
                            zs1[d] = copy(zsend, zrecv, z_s.at[d],
                                          z_r.at[d], partner8)

        r_own0 = lax.rem(p + 1, NP)
        r_own1 = lax.rem(p - 1 + NP, NP)

        e0, e1 = zs1[0], zs1[1]
        e0.wait_recv()
        own0[pl.ds(keep_off, half), :] = (
            own0[pl.ds(keep_off, half), :] + zs1_recv0[...].astype(jnp.float32)
        )
        zx_send0[...] = own0[pl.ds(keep_off, half), :].astype(jnp.bfloat16)
        f0 = copy(zx_send0, zx_recv0, z_s.at[2], z_r.at[2], partner16)
        e1.wait_recv()
        own1[pl.ds(keep_off, half), :] = (
            own1[pl.ds(keep_off, half), :] + zs1_recv1[...].astype(jnp.float32)
        )
        zx_send1[...] = own1[pl.ds(keep_off, half), :].astype(jnp.bfloat16)
        f1 = copy(zx_send1, zx_recv1, z_s.at[3], z_r.at[3], partner16)
        cbufs = (c_buf0, c_buf1)
        csems = ((c_s0, c_r0), (c_s1, c_r1))
        c_rows = (keep_off, send_off)
        zg1s = ((zg1_send0, zg1_recv0), (zg1_send1, zg1_recv1))

        def c_start(d, h, j):
            src = zg1s[d][j] if h == 0 else (
                cbufs[d].at[h, pl.ds(c_rows[j], subrows)]
            )
            return copy(
                src,
                cbufs[d].at[h + 1, pl.ds(c_rows[j], subrows)],
                csems[d][0].at[h, j], csems[d][1].at[h, j], nbr_of[d],
            )

        crec = {}
        f0.wait_recv()
        finh0 = jnp.maximum(
            own0[pl.ds(keep_off, half), :] + zx_recv0[...].astype(jnp.float32),
            0.0,
        )
        zg1_send0[...] = finh0.astype(jnp.bfloat16)
        h0 = copy(zg1_send0, zg1_recv0, z_s.at[4], z_r.at[4], partner8)
        crec[(0, 0, 0)] = c_start(0, 0, 0)
        f1.wait_recv()
        finh1 = jnp.maximum(
            own1[pl.ds(keep_off, half), :] + zx_recv1[...].astype(jnp.float32),
            0.0,
        )
        zg1_send1[...] = finh1.astype(jnp.bfloat16)
        h1 = copy(zg1_send1, zg1_recv1, z_s.at[5], z_r.at[5], partner8)
        crec[(1, 0, 0)] = c_start(1, 0, 0)
        out_ref[pl.ds(r_own0 * piece + keep_off, half), 0:nh] = (
            zg1_send0[...].astype(jnp.float32)
        )
        out_ref[pl.ds(r_own1 * piece + keep_off, half), nh:n] = (
            zg1_send1[...].astype(jnp.float32)
        )
        h0.wait_recv()
        crec[(0, 0, 1)] = c_start(0, 0, 1)
        h1.wait_recv()
        crec[(1, 0, 1)] = c_start(1, 0, 1)
        out_ref[pl.ds(r_own0 * piece + send_off, half), 0:nh] = (
            zg1_recv0[...].astype(jnp.float32)
        )
        out_ref[pl.ds(r_own1 * piece + send_off, half), nh:n] = (
            zg1_recv1[...].astype(jnp.float32)
        )

        for h in range(NP - 1):
            g0i = lax.rem(p - h + NP, NP)
            g1i = lax.rem(p + h, NP)
            gs = (g0i, g1i)
            for j in range(NSUB):
                for d in range(2):
                    crec[(d, h, j)].wait_recv()
                    if h < NP - 2:
                        crec[(d, h + 1, j)] = c_start(d, h + 1, j)
                    lo, hi = (0, nh) if d == 0 else (nh, n)
                    out_ref[pl.ds(gs[d] * piece + c_rows[j], subrows),
                            lo:hi] = (
                        cbufs[d][h + 1, pl.ds(c_rows[j], subrows), :]
                        .astype(jnp.float32)
                    )

        for rdma in pending:
            rdma.wait_send()

    return pl.pallas_call(
        body,
        out_shape=jax.ShapeDtypeStruct((m, n), jnp.float32),
        in_specs=[
            pl.BlockSpec(memory_space=pltpu.VMEM),
            pl.BlockSpec(memory_space=pltpu.VMEM),
        ],
        out_specs=pl.BlockSpec(memory_space=pltpu.VMEM),
        scratch_shapes=[
            pltpu.VMEM((m, nh), jnp.float32),
            pltpu.VMEM((m, nh), jnp.float32),
            pltpu.VMEM((NP, NSUB, piece // NSUB, nh), jnp.bfloat16),
            pltpu.VMEM((NP, NSUB, piece // NSUB, nh), jnp.bfloat16),
            pltpu.VMEM((NP, piece, nh), jnp.bfloat16),
            pltpu.VMEM((NP, piece, nh), jnp.bfloat16),
            pltpu.VMEM((piece, nh), jnp.float32),
            pltpu.VMEM((piece, nh), jnp.float32),
            pltpu.VMEM((piece // 2, nh), jnp.bfloat16),
            pltpu.VMEM((piece // 2, nh), jnp.bfloat16),
            pltpu.VMEM((piece // 2, nh), jnp.bfloat16),
            pltpu.VMEM((piece // 2, nh), jnp.bfloat16),
            pltpu.VMEM((piece // 2, nh), jnp.bfloat16),
            pltpu.VMEM((piece // 2, nh), jnp.bfloat16),
            pltpu.VMEM((piece // 2, nh), jnp.bfloat16),
            pltpu.VMEM((piece // 2, nh), jnp.bfloat16),
            pltpu.VMEM((piece // 2, nh), jnp.bfloat16),
            pltpu.VMEM((piece // 2, nh), jnp.bfloat16),
            pltpu.VMEM((piece // 2, nh), jnp.bfloat16),
            pltpu.VMEM((piece // 2, nh), jnp.bfloat16),
            pltpu.SemaphoreType.DMA((NP - 1, NSUB)),
            pltpu.SemaphoreType.DMA((NP - 1, NSUB)),
            pltpu.SemaphoreType.DMA((NP - 1, NSUB)),
            pltpu.SemaphoreType.DMA((NP - 1, NSUB)),
            pltpu.SemaphoreType.DMA((8,)),
            pltpu.SemaphoreType.DMA((8,)),
            pltpu.SemaphoreType.DMA((NP - 1, NSUB)),
            pltpu.SemaphoreType.DMA((NP - 1, NSUB)),
            pltpu.SemaphoreType.DMA((NP - 1, NSUB)),
            pltpu.SemaphoreType.DMA((NP - 1, NSUB)),
        ],
        compiler_params=pltpu.CompilerParams(
            collective_id=0,
            vmem_limit_bytes=100 * 1024 * 1024,
        ),
    )(x, w_mat)
